# Initial kernel scaffold; baseline (speedup 1.0000x reference)
import jax
import jax.numpy as jnp
from jax.experimental import pallas as pl


def kernel(hidden_states, gate_w, Wg, Wu, Wd, Sg, Su, Sd, seg_w):
    raise NotImplementedError("write your pallas kernel here")


# speedup vs baseline: 1.0096x; 1.0096x over previous
import functools

import jax
import jax.numpy as jnp
from jax.experimental import pallas as pl
from jax.experimental.pallas import tpu as pltpu

F32 = jnp.float32
BF16 = jnp.bfloat16
_HIGH = jax.lax.Precision.HIGHEST


def _router_body(x_ref, gw_ref, comb_ref):
    x = x_ref[...]
    gw = gw_ref[...]
    T = x.shape[0]
    logits = jax.lax.dot_general(x, gw, (((1,), (1,)), ((), ())),
                                 preferred_element_type=F32)
    lane = jax.lax.broadcasted_iota(jnp.int32, (T, 128), 1)
    valid = lane < 8
    logits = jnp.where(valid, logits, -1e30)
    m = jnp.max(logits, axis=1, keepdims=True)
    p = jnp.exp(logits - m)
    p = p / jnp.sum(p, axis=1, keepdims=True)
    w1 = jnp.max(p, axis=1, keepdims=True)
    i1 = jnp.min(jnp.where(p == w1, lane, 999), axis=1, keepdims=True)
    p2 = jnp.where(lane == i1, -1.0, p)
    w2 = jnp.max(p2, axis=1, keepdims=True)
    i2 = jnp.min(jnp.where(p2 == w2, lane, 999), axis=1, keepdims=True)
    comb = jnp.where(lane == i1, w1, 0.0) + jnp.where(lane == i2, w2, 0.0)
    comb_ref[...] = comb


def _experts_body(comb_ref, x_ref, wg_ref, wu_ref, wd_ref, out_ref):
    e = pl.program_id(1)
    x = x_ref[...]
    wg = wg_ref[0]
    wu = wu_ref[0]
    wd = wd_ref[0]
    g = jax.lax.dot_general(x, wg, (((1,), (1,)), ((), ())),
                            preferred_element_type=F32)
    u = jax.lax.dot_general(x, wu, (((1,), (1,)), ((), ())),
                            preferred_element_type=F32)
    h = (g * jax.nn.sigmoid(g) * u).astype(BF16)
    y = jax.lax.dot_general(h, wd, (((1,), (1,)), ((), ())),
                            preferred_element_type=F32)
    lane = jax.lax.broadcasted_iota(jnp.int32, comb_ref.shape, 1)
    w = jnp.sum(jnp.where(lane == e, comb_ref[...], 0.0), axis=1,
                keepdims=True)
    y = y * w

    @pl.when(e == 0)
    def _init():
        out_ref[...] = y

    @pl.when(e > 0)
    def _acc():
        out_ref[...] += y


def _shared_body(x_ref, sg_ref, su_ref, sd_ref, segw_ref, moe_ref, out_ref):
    j = pl.program_id(1)
    nj = pl.num_programs(1)
    x = x_ref[...]
    g = jax.lax.dot_general(x, sg_ref[...], (((1,), (1,)), ((), ())),
                            preferred_element_type=F32)
    u = jax.lax.dot_general(x, su_ref[...], (((1,), (1,)), ((), ())),
                            preferred_element_type=F32)
    h = (g * jax.nn.sigmoid(g) * u).astype(BF16)
    y = jax.lax.dot_general(h, sd_ref[...], (((1,), (1,)), ((), ())),
                            preferred_element_type=F32)

    @pl.when(j == 0)
    def _init():
        out_ref[...] = y

    @pl.when(j > 0)
    def _acc():
        out_ref[...] += y

    @pl.when(j == nj - 1)
    def _fin():
        sl = jax.lax.dot_general(x, segw_ref[...], (((1,), (1,)), ((), ())),
                                 preferred_element_type=F32)
        lane = jax.lax.broadcasted_iota(jnp.int32, sl.shape, 1)
        gate = jnp.sum(jnp.where(lane == 0, jax.nn.sigmoid(sl), 0.0),
                       axis=1, keepdims=True)
        out_ref[...] = moe_ref[...] + gate * out_ref[...]


def kernel(hidden_states, gate_w, Wg, Wu, Wd, Sg, Su, Sd, seg_w):
    b, s, d = hidden_states.shape
    x = hidden_states.reshape(-1, d)
    T, D = x.shape
    E, DFF, _ = Wg.shape
    DFF_S = Sg.shape[0]

    gw_pad = jnp.zeros((128, D), F32).at[:E].set(gate_w)
    comb = pl.pallas_call(
        _router_body,
        out_shape=jax.ShapeDtypeStruct((T, 128), F32),
    )(x, gw_pad)

    xbf = x.astype(BF16)
    BT = min(512, T)
    moe = pl.pallas_call(
        _experts_body,
        grid=(T // BT, E),
        in_specs=[
            pl.BlockSpec((BT, 128), lambda i, e: (i, 0)),
            pl.BlockSpec((BT, D), lambda i, e: (i, 0)),
            pl.BlockSpec((1, DFF, D), lambda i, e: (e, 0, 0)),
            pl.BlockSpec((1, DFF, D), lambda i, e: (e, 0, 0)),
            pl.BlockSpec((1, D, DFF), lambda i, e: (e, 0, 0)),
        ],
        out_specs=pl.BlockSpec((BT, D), lambda i, e: (i, 0)),
        out_shape=jax.ShapeDtypeStruct((T, D), F32),
    )(comb, xbf, Wg.astype(BF16), Wu.astype(BF16), Wd.astype(BF16))

    BF = 512 if DFF_S % 512 == 0 else DFF_S
    segw_pad = jnp.zeros((128, D), BF16).at[:1].set(seg_w.astype(BF16))
    out = pl.pallas_call(
        _shared_body,
        grid=(T // BT, DFF_S // BF),
        in_specs=[
            pl.BlockSpec((BT, D), lambda i, j: (i, 0)),
            pl.BlockSpec((BF, D), lambda i, j: (j, 0)),
            pl.BlockSpec((BF, D), lambda i, j: (j, 0)),
            pl.BlockSpec((D, BF), lambda i, j: (0, j)),
            pl.BlockSpec((128, D), lambda i, j: (0, 0)),
            pl.BlockSpec((BT, D), lambda i, j: (i, 0)),
        ],
        out_specs=pl.BlockSpec((BT, D), lambda i, j: (i, 0)),
        out_shape=jax.ShapeDtypeStruct((T, D), F32),
    )(xbf, Sg.astype(BF16), Su.astype(BF16), Sd.astype(BF16), segw_pad, moe)

    return out.reshape(b, s, d)
